# Initial kernel scaffold; baseline (speedup 1.0000x reference)
import jax
import jax.numpy as jnp
from jax.experimental import pallas as pl


def kernel(x, edges, weight, W1, b1, gn_w, gn_b, gn_ms, Wg, att_src, att_dst, bg):
    raise NotImplementedError("write your pallas kernel here")


# speedup vs baseline: 15.3008x; 15.3008x over previous
import functools

import jax
import jax.numpy as jnp
from jax import lax
from jax.experimental import pallas as pl
from jax.experimental.pallas import tpu as pltpu
from jax.experimental.pallas import tpu_sc as plsc

N = 10000
E = 320000
D = 128
NC = 2
NS = 16
NW = NC * NS
NP = 10240
STR = NP // NS
EPT = E // NW
CH = 80
NCH = EPT // CH
BR = 512
NB = NP // BR
HN = NP // 2
ACC_R = HN + 128
OSTR = HN // NS
ASTR = ACC_R // NS
DUMMY = HN
EPTP = EPT + 96
PSH = 13
ECH = 2000
NEC = EPT // ECH
F32 = jnp.float32
I32 = jnp.int32


def _mesh():
    return plsc.VectorSubcoreMesh(
        core_axis_name="c", subcore_axis_name="s", num_cores=NC, num_subcores=NS)


def _seg_combine(kbuf, vbuf, k16, v16, op):
    ks, vs = plsc.sort_key_val(k16, v16)
    iota = lax.iota(I32, 16)
    kbuf[...] = ks
    val = vs
    for s in (1, 2, 4, 8):
        vbuf[...] = val
        idx = jnp.maximum(iota - s, 0)
        kp = plsc.load_gather(kbuf, [idx])
        vp = plsc.load_gather(vbuf, [idx])
        same = (kp == ks) & (iota >= s)
        val = jnp.where(same, op(val, vp), val)
    kn = plsc.load_gather(kbuf, [jnp.minimum(iota + 1, 15)])
    endmask = (kn != ks) | (iota == 15)
    return ks, val, endmask


def _combine_tiles(loc, shared, stripebuf, accb, out_ref, cid, sid, op):
    pltpu.sync_copy(loc, shared.at[sid])
    plsc.subcore_barrier()
    base = sid * STR
    pltpu.sync_copy(shared.at[:, pl.ds(base, STR)], stripebuf)

    def body(i, _):
        v = stripebuf[0, pl.ds(i * 16, 16)]
        for k in range(1, NS):
            v = op(v, stripebuf[k, pl.ds(i * 16, 16)])
        accb[pl.ds(i * 16, 16)] = v
        return 0

    lax.fori_loop(0, STR // 16, body, 0)
    pltpu.sync_copy(accb, out_ref.at[cid, pl.ds(base, STR)])


def _fill_np(loc, value):
    v16 = jnp.full((16,), value, F32)

    def body(i, _):
        loc[pl.ds(i * 16, 16)] = v16
        return 0

    lax.fori_loop(0, NP // 16, body, 0)


def _zero_rows(rows):
    z16 = jnp.zeros((16,), F32)

    def body(r, _):
        for c in range(D // 16):
            rows[r, pl.ds(c * 16, 16)] = z16
        return 0

    lax.fori_loop(0, CH, body, 0)


def _zero_acc_stripe(rows, acc_sh, sid):
    _zero_rows(rows)
    base = sid * ASTR
    for k in range(ASTR // CH):
        pltpu.sync_copy(rows, acc_sh.at[pl.ds(base + k * CH, CH), :])
    rem = ASTR % CH
    if rem:
        pltpu.sync_copy(rows.at[pl.ds(0, rem), :],
                        acc_sh.at[pl.ds(base + (ASTR // CH) * CH, rem), :])


def _scale_rows(rows, get_scale):
    def body(r, _):
        b = get_scale(r)
        for c in range(D // 16):
            rows[r, pl.ds(c * 16, 16)] = rows[r, pl.ds(c * 16, 16)] * b
        return 0

    lax.fori_loop(0, CH, body, 0)


def _compact_store(cbuf, pendc, half, cnt, pend, d16, s16,
                   sbuf=None, pends=None, sc16=None):
    rel = d16 - half * HN
    ok = (rel >= 0) & (rel < HN)
    p = (s16 << PSH) | jnp.where(ok, rel, 0)
    plsc.store_compressed(pendc.at[half, pl.ds(pend, 16)], p, mask=ok)
    if sbuf is not None:
        plsc.store_compressed(pends.at[half, pl.ds(pend, 16)], sc16, mask=ok)
    ntot = pend + jnp.max(plsc.all_reduce_population_count(ok))
    flush = ntot >= 16

    @pl.when(flush)
    def _():
        cbuf[half, pl.ds(cnt, 16)] = pendc[half, pl.ds(0, 16)]
        pendc[half, pl.ds(0, 16)] = pendc[half, pl.ds(16, 16)]
        if sbuf is not None:
            sbuf[half, pl.ds(cnt, 16)] = pends[half, pl.ds(0, 16)]
            pends[half, pl.ds(0, 16)] = pends[half, pl.ds(16, 16)]

    return jnp.where(flush, cnt + 16, cnt), jnp.where(flush, ntot - 16, ntot)


def _finalize_compacted(cbuf, pendc, half, cnt, pend, sbuf=None, pends=None):
    iota = lax.iota(I32, 16)
    v = jnp.where(iota < pend, pendc[half, pl.ds(0, 16)], DUMMY)
    cbuf[half, pl.ds(cnt, 16)] = v
    if sbuf is not None:
        sv = jnp.where(iota < pend, pends[half, pl.ds(0, 16)], 0.0)
        sbuf[half, pl.ds(cnt, 16)] = sv
    total = cnt + 16
    target = ((total + CH - 1) // CH) * CH
    padp = jnp.full((16,), DUMMY, I32)
    padz = jnp.zeros((16,), F32)

    def body(i, _):
        cbuf[half, pl.ds(total + i * 16, 16)] = padp
        if sbuf is not None:
            sbuf[half, pl.ds(total + i * 16, 16)] = padz
        return 0

    lax.fori_loop(0, (target - total) // 16, body, 0)
    return target // CH


def _unpack_chunk(cbuf, half, base, srcb, dstb, g):
    p16 = cbuf[half, pl.ds(base + g * 16, 16)]
    s16 = p16 >> PSH
    r16 = p16 & ((1 << PSH) - 1)
    srcb[pl.ds(g * 16, 16)] = s16
    dstb[pl.ds(g * 16, 16)] = r16
    return s16, r16


def _copy_out_stripe(out_hbm, acc_sh, cid, half, sid):
    pltpu.sync_copy(acc_sh.at[pl.ds(sid * OSTR, OSTR), :],
                    out_hbm.at[cid, half, pl.ds(sid * OSTR, OSTR), :])


def _deg_body(dst_hbm, w_hbm, out_hbm, dstv, wv, loc, kbuf, vbuf,
              shared, stripebuf, accb):
    cid = lax.axis_index("c")
    sid = lax.axis_index("s")
    wid = cid * NS + sid
    pltpu.sync_copy(dst_hbm.at[wid], dstv)
    pltpu.sync_copy(w_hbm.at[wid], wv)
    _fill_np(loc, 0.0)

    def body(j, _):
        for g in range(CH // 16):
            d16 = dstv[j, pl.ds(g * 16, 16)]
            v16 = wv[j, pl.ds(g * 16, 16)]
            ks, tot, em = _seg_combine(kbuf, vbuf, d16, v16, lambda a, b: a + b)
            plsc.addupdate_scatter(loc, [ks], tot, mask=em)
        return 0

    lax.fori_loop(0, NCH, body, 0)
    _combine_tiles(loc, shared, stripebuf, accb, out_hbm, cid, sid,
                   lambda a, b: a + b)


def _deg_call(dst_r, w_r):
    k = functools.partial(
        pl.kernel,
        out_type=jax.ShapeDtypeStruct((NC, NP), F32),
        mesh=_mesh(),
        compiler_params=pltpu.CompilerParams(needs_layout_passes=False),
        scratch_types=[
            pltpu.VMEM((NCH, CH), I32),
            pltpu.VMEM((NCH, CH), F32),
            pltpu.VMEM((NP,), F32),
            pltpu.VMEM((16,), I32),
            pltpu.VMEM((16,), F32),
            pltpu.VMEM_SHARED((NS, NP), F32),
            pltpu.VMEM((NS, STR), F32),
            pltpu.VMEM((STR,), F32),
        ],
    )(_deg_body)
    return k(dst_r, w_r)


def _gcn_body(src_hbm, dst_hbm, w_hbm, dis_hbm, h_hbm, out_hbm,
              srcc, dstc, wc, disv, cbuf, sbuf, pendc, pends, rows,
              srcb, dstb, sem, acc_sh):
    cid = lax.axis_index("c")
    sid = lax.axis_index("s")
    wid = cid * NS + sid
    pltpu.sync_copy(dis_hbm, disv)

    def outer(r, carry):
        base = wid * EPT + r * ECH
        pltpu.sync_copy(src_hbm.at[pl.ds(base, ECH)], srcc)
        pltpu.sync_copy(dst_hbm.at[pl.ds(base, ECH)], dstc)
        pltpu.sync_copy(w_hbm.at[pl.ds(base, ECH)], wc)

        def cb(g, carry):
            c0, p0, c1, p1 = carry
            s16 = srcc[pl.ds(g * 16, 16)]
            d16 = dstc[pl.ds(g * 16, 16)]
            sc = wc[pl.ds(g * 16, 16)] * plsc.load_gather(disv, [s16])
            c0, p0 = _compact_store(cbuf, pendc, 0, c0, p0, d16, s16,
                                    sbuf, pends, sc)
            c1, p1 = _compact_store(cbuf, pendc, 1, c1, p1, d16, s16,
                                    sbuf, pends, sc)
            return c0, p0, c1, p1

        return lax.fori_loop(0, ECH // 16, cb, carry)

    z = jnp.int32(0)
    c0, p0, c1, p1 = lax.fori_loop(0, NEC, outer, (z, z, z, z))
    for half, nch in ((0, _finalize_compacted(cbuf, pendc, 0, c0, p0,
                                              sbuf, pends)),
                      (1, _finalize_compacted(cbuf, pendc, 1, c1, p1,
                                              sbuf, pends))):
        _zero_acc_stripe(rows, acc_sh, sid)
        plsc.subcore_barrier()
        h16 = jnp.full((16,), half, I32)

        def body(j, _):
            base = j * CH
            for g in range(CH // 16):
                _unpack_chunk(cbuf, half, base, srcb, dstb, g)
            pltpu.async_copy(h_hbm.at[srcb], rows, sem).wait()
            _scale_rows(rows, lambda r: plsc.load_gather(
                sbuf, [h16, jnp.broadcast_to(base + r, (16,))]))
            pltpu.sync_copy(rows, acc_sh.at[dstb], add=True)
            return 0

        lax.fori_loop(0, nch, body, 0)
        plsc.subcore_barrier()
        _copy_out_stripe(out_hbm, acc_sh, cid, half, sid)


def _gcn_call(src, dst, w, dis, h):
    k = functools.partial(
        pl.kernel,
        out_type=jax.ShapeDtypeStruct((NC, 2, HN, D), F32),
        mesh=_mesh(),
        compiler_params=pltpu.CompilerParams(needs_layout_passes=False),
        scratch_types=[
            pltpu.VMEM((ECH,), I32),
            pltpu.VMEM((ECH,), I32),
            pltpu.VMEM((ECH,), F32),
            pltpu.VMEM((NP,), F32),
            pltpu.VMEM((2, EPTP), I32),
            pltpu.VMEM((2, EPTP), F32),
            pltpu.VMEM((2, 48), I32),
            pltpu.VMEM((2, 48), F32),
            pltpu.VMEM((CH, D), F32),
            pltpu.VMEM((CH,), I32),
            pltpu.VMEM((CH,), I32),
            pltpu.SemaphoreType.DMA,
            pltpu.VMEM_SHARED((ACC_R, D), F32),
        ],
    )(_gcn_body)
    return k(src, dst, w, dis, h)


def _amax_body(src_hbm, dst_hbm, asrc_hbm, adst_hbm, out_hbm,
               srcv, dstv, asv, adv, loc, kbuf, vbuf, shared, stripebuf, accb):
    cid = lax.axis_index("c")
    sid = lax.axis_index("s")
    wid = cid * NS + sid
    pltpu.sync_copy(src_hbm.at[wid], srcv)
    pltpu.sync_copy(dst_hbm.at[wid], dstv)
    pltpu.sync_copy(asrc_hbm, asv)
    pltpu.sync_copy(adst_hbm, adv)
    _fill_np(loc, -1e30)

    def body(j, _):
        for g in range(CH // 16):
            s16 = srcv[j, pl.ds(g * 16, 16)]
            d16 = dstv[j, pl.ds(g * 16, 16)]
            a = plsc.load_gather(asv, [s16]) + plsc.load_gather(adv, [d16])
            a = jnp.where(a > 0, a, 0.2 * a)
            ks, tot, em = _seg_combine(kbuf, vbuf, d16, a, jnp.maximum)
            cur = plsc.load_gather(loc, [ks])
            plsc.store_scatter(loc, [ks], jnp.maximum(cur, tot), mask=em)
        return 0

    lax.fori_loop(0, NCH, body, 0)
    _combine_tiles(loc, shared, stripebuf, accb, out_hbm, cid, sid, jnp.maximum)


def _amax_call(src_r, dst_r, asrc, adst):
    k = functools.partial(
        pl.kernel,
        out_type=jax.ShapeDtypeStruct((NC, NP), F32),
        mesh=_mesh(),
        compiler_params=pltpu.CompilerParams(needs_layout_passes=False),
        scratch_types=[
            pltpu.VMEM((NCH, CH), I32),
            pltpu.VMEM((NCH, CH), I32),
            pltpu.VMEM((NP,), F32),
            pltpu.VMEM((NP,), F32),
            pltpu.VMEM((NP,), F32),
            pltpu.VMEM((16,), I32),
            pltpu.VMEM((16,), F32),
            pltpu.VMEM_SHARED((NS, NP), F32),
            pltpu.VMEM((NS, STR), F32),
            pltpu.VMEM((STR,), F32),
        ],
    )(_amax_body)
    return k(src_r, dst_r, asrc, adst)


def _gat_body(src_hbm, dst_hbm, asrc_hbm, adst_hbm, amax_hbm, h2_hbm,
              outacc_hbm, outden_hbm,
              srcc, dstc, asv, adv, amv, cbuf, pendc, rows, srcb, dstb,
              scaleb, denloc, kbuf, vbuf, sem, acc_sh):
    cid = lax.axis_index("c")
    sid = lax.axis_index("s")
    wid = cid * NS + sid
    pltpu.sync_copy(asrc_hbm, asv)
    pltpu.sync_copy(adst_hbm, adv)
    pltpu.sync_copy(amax_hbm, amv)
    _fill_np(denloc, 0.0)

    def outer(r, carry):
        base = wid * EPT + r * ECH
        pltpu.sync_copy(src_hbm.at[pl.ds(base, ECH)], srcc)
        pltpu.sync_copy(dst_hbm.at[pl.ds(base, ECH)], dstc)

        def cb(g, carry):
            c0, p0, c1, p1 = carry
            s16 = srcc[pl.ds(g * 16, 16)]
            d16 = dstc[pl.ds(g * 16, 16)]
            c0, p0 = _compact_store(cbuf, pendc, 0, c0, p0, d16, s16)
            c1, p1 = _compact_store(cbuf, pendc, 1, c1, p1, d16, s16)
            return c0, p0, c1, p1

        return lax.fori_loop(0, ECH // 16, cb, carry)

    z = jnp.int32(0)
    c0, p0, c1, p1 = lax.fori_loop(0, NEC, outer, (z, z, z, z))
    for half, nch in ((0, _finalize_compacted(cbuf, pendc, 0, c0, p0)),
                      (1, _finalize_compacted(cbuf, pendc, 1, c1, p1))):
        _zero_acc_stripe(rows, acc_sh, sid)
        plsc.subcore_barrier()

        def body(j, _):
            base = j * CH
            for g in range(CH // 16):
                s16, r16 = _unpack_chunk(cbuf, half, base, srcb, dstb, g)
                pad = r16 == DUMMY
                dabs = jnp.where(pad, 0, r16 + half * HN)
                a = (plsc.load_gather(asv, [s16])
                     + plsc.load_gather(adv, [dabs]))
                a = jnp.where(a > 0, a, 0.2 * a)
                e = jnp.exp(a - plsc.load_gather(amv, [dabs]))
                e = jnp.where(pad, 0.0, e)
                scaleb[pl.ds(g * 16, 16)] = e
                ks, tot, em = _seg_combine(kbuf, vbuf, dabs, e,
                                           lambda x, y: x + y)
                plsc.addupdate_scatter(denloc, [ks], tot, mask=em)
            pltpu.async_copy(h2_hbm.at[srcb], rows, sem).wait()
            _scale_rows(rows, lambda r: plsc.load_gather(
                scaleb, [jnp.broadcast_to(r, (16,))]))
            pltpu.sync_copy(rows, acc_sh.at[dstb], add=True)
            return 0

        lax.fori_loop(0, nch, body, 0)
        plsc.subcore_barrier()
        _copy_out_stripe(outacc_hbm, acc_sh, cid, half, sid)
    pltpu.sync_copy(denloc, outden_hbm.at[cid, sid])


def _gat_call(src, dst, asrc, adst, amax, h2):
    k = functools.partial(
        pl.kernel,
        out_type=[jax.ShapeDtypeStruct((NC, 2, HN, D), F32),
                  jax.ShapeDtypeStruct((NC, NS, NP), F32)],
        mesh=_mesh(),
        compiler_params=pltpu.CompilerParams(needs_layout_passes=False),
        scratch_types=[
            pltpu.VMEM((ECH,), I32),
            pltpu.VMEM((ECH,), I32),
            pltpu.VMEM((NP,), F32),
            pltpu.VMEM((NP,), F32),
            pltpu.VMEM((NP,), F32),
            pltpu.VMEM((2, EPTP), I32),
            pltpu.VMEM((2, 48), I32),
            pltpu.VMEM((CH, D), F32),
            pltpu.VMEM((CH,), I32),
            pltpu.VMEM((CH,), I32),
            pltpu.VMEM((CH,), F32),
            pltpu.VMEM((NP,), F32),
            pltpu.VMEM((16,), I32),
            pltpu.VMEM((16,), F32),
            pltpu.SemaphoreType.DMA,
            pltpu.VMEM_SHARED((ACC_R, D), F32),
        ],
    )(_gat_body)
    return k(src, dst, asrc, adst, amax, h2)


def _mm_body(x_ref, w_ref, o_ref):
    o_ref[...] = jnp.dot(x_ref[...], w_ref[...],
                         preferred_element_type=jnp.float32)


def _matmul(xp, W):
    return pl.pallas_call(
        _mm_body,
        grid=(NB,),
        in_specs=[pl.BlockSpec((BR, D), lambda i: (i, 0)),
                  pl.BlockSpec((D, D), lambda i: (0, 0))],
        out_specs=pl.BlockSpec((BR, D), lambda i: (i, 0)),
        out_shape=jax.ShapeDtypeStruct((NP, D), F32),
    )(xp, W)


def _tc2_body(x_ref, h_ref, a0_ref, a1_ref, dis_ref,
              b1_ref, gnw_ref, gnb_ref, gnms_ref, Wg_ref, as_ref, ad_ref,
              x1_ref, h2_ref, asv_ref, adv_ref, sum_ref, sq_ref):
    p = pl.program_id(0)
    i = pl.program_id(1)
    disv = dis_ref[...][:, None]
    acc = a0_ref[...] + a1_ref[...]
    t = disv * (acc + disv * h_ref[...]) + b1_ref[...][None, :]
    rid = i * BR + lax.broadcasted_iota(I32, (BR, 1), 0)
    t = jnp.where(rid < N, t, 0.0)

    @pl.when(p == 0)
    def _():
        @pl.when(i == 0)
        def _():
            sum_ref[...] = jnp.zeros_like(sum_ref)
            sq_ref[...] = jnp.zeros_like(sq_ref)

        sum_ref[...] += jnp.sum(t, axis=0, keepdims=True)
        sq_ref[...] += jnp.sum(t * t, axis=0, keepdims=True)

    @pl.when(p == 1)
    def _():
        ms = gnms_ref[...][None, :]
        mean = sum_ref[...] / N
        e2 = sq_ref[...] / N
        var = e2 - (2.0 - ms) * ms * mean * mean
        gn = gnw_ref[...][None, :] * (t - ms * mean) / jnp.sqrt(var + 1e-5) \
            + gnb_ref[...][None, :]
        l = jnp.where(gn > 0, gn, 0.01 * gn)
        x1 = x_ref[...] + l
        x1_ref[...] = x1
        h2 = jnp.dot(x1, Wg_ref[...], preferred_element_type=jnp.float32)
        h2_ref[...] = h2
        asv_ref[...] = jnp.sum(h2 * as_ref[...][None, :], axis=1)
        adv_ref[...] = jnp.sum(h2 * ad_ref[...][None, :], axis=1)


def _tc2_call(xp, h, a0, a1, dis, b1, gn_w, gn_b, gn_ms, Wg,
              att_src, att_dst):
    mat = lambda: pl.BlockSpec((BR, D), lambda p, i: (i, 0))
    vec = lambda: pl.BlockSpec((BR,), lambda p, i: (i,))
    dvec = lambda: pl.BlockSpec((D,), lambda p, i: (0,))
    return pl.pallas_call(
        _tc2_body,
        grid=(2, NB),
        in_specs=[mat(), mat(), mat(), mat(), vec(),
                  dvec(), dvec(), dvec(), dvec(),
                  pl.BlockSpec((D, D), lambda p, i: (0, 0)), dvec(), dvec()],
        out_specs=[mat(), mat(), vec(), vec()],
        out_shape=[jax.ShapeDtypeStruct((NP, D), F32),
                   jax.ShapeDtypeStruct((NP, D), F32),
                   jax.ShapeDtypeStruct((NP,), F32),
                   jax.ShapeDtypeStruct((NP,), F32)],
        scratch_shapes=[pltpu.VMEM((1, D), F32), pltpu.VMEM((1, D), F32)],
    )(xp, h, a0, a1, dis, b1, gn_w, gn_b, gn_ms, Wg, att_src, att_dst)


def _tc3_body(x1_ref, h2_ref, g0_ref, g1_ref, es_ref,
              den_ref, bg_ref, gnw_ref, gnb_ref, gnms_ref, out_ref,
              sum_ref, sq_ref):
    p = pl.program_id(0)
    i = pl.program_id(1)
    esv = es_ref[...][:, None]
    denv = (jnp.sum(den_ref[...], axis=0) + es_ref[...])[:, None]
    acc = g0_ref[...] + g1_ref[...]
    t = (acc + h2_ref[...] * esv) / (denv + 1e-16) + bg_ref[...][None, :]
    rid = i * BR + lax.broadcasted_iota(I32, (BR, 1), 0)
    t = jnp.where(rid < N, t, 0.0)

    @pl.when(p == 0)
    def _():
        @pl.when(i == 0)
        def _():
            sum_ref[...] = jnp.zeros_like(sum_ref)
            sq_ref[...] = jnp.zeros_like(sq_ref)

        sum_ref[...] += jnp.sum(t, axis=0, keepdims=True)
        sq_ref[...] += jnp.sum(t * t, axis=0, keepdims=True)

    @pl.when(p == 1)
    def _():
        ms = gnms_ref[...][None, :]
        mean = sum_ref[...] / N
        e2 = sq_ref[...] / N
        var = e2 - (2.0 - ms) * ms * mean * mean
        gn = gnw_ref[...][None, :] * (t - ms * mean) / jnp.sqrt(var + 1e-5) \
            + gnb_ref[...][None, :]
        l = jnp.where(gn > 0, gn, 0.01 * gn)
        out_ref[...] = x1_ref[...] + l


def _tc3_call(x1, h2, g0, g1, es, den, bg, gn_w, gn_b, gn_ms):
    mat = lambda: pl.BlockSpec((BR, D), lambda p, i: (i, 0))
    vec = lambda: pl.BlockSpec((BR,), lambda p, i: (i,))
    dvec = lambda: pl.BlockSpec((D,), lambda p, i: (0,))
    return pl.pallas_call(
        _tc3_body,
        grid=(2, NB),
        in_specs=[mat(), mat(), mat(), mat(), vec(),
                  pl.BlockSpec((NW, BR), lambda p, i: (0, i)),
                  dvec(), dvec(), dvec(), dvec()],
        out_specs=mat(),
        out_shape=jax.ShapeDtypeStruct((NP, D), F32),
        scratch_shapes=[pltpu.VMEM((1, D), F32), pltpu.VMEM((1, D), F32)],
    )(x1, h2, g0, g1, es, den, bg, gn_w, gn_b, gn_ms)


def kernel(x, edges, weight, W1, b1, gn_w, gn_b, gn_ms, Wg, att_src, att_dst,
           bg):
    xp = jnp.zeros((NP, D), F32).at[:N].set(x)
    src_r = edges[0].reshape(NW, NCH, CH)
    dst_r = edges[1].reshape(NW, NCH, CH)
    w_r = weight.reshape(NW, NCH, CH)

    h = _matmul(xp, W1)
    degp = _deg_call(dst_r, w_r)
    deg = degp[0] + degp[1] + 1.0
    dis = jnp.where(deg > 0, 1.0 / jnp.sqrt(deg), 0.0)
    accA = _gcn_call(edges[0], edges[1], weight, dis, h).reshape(NC, NP, D)
    x1, h2, asv, adv = _tc2_call(xp, h, accA[0], accA[1], dis, b1,
                                 gn_w, gn_b, gn_ms, Wg, att_src, att_dst)

    amaxp = _amax_call(src_r, dst_r, asv, adv)
    aself = asv + adv
    aself = jnp.where(aself > 0, aself, 0.2 * aself)
    amax = jnp.maximum(jnp.maximum(amaxp[0], amaxp[1]), aself)
    accG, denp = _gat_call(edges[0], edges[1], asv, adv, amax, h2)
    accG = accG.reshape(NC, NP, D)
    es = jnp.exp(aself - amax)
    out = _tc3_call(x1, h2, accG[0], accG[1], es, denp.reshape(NW, NP), bg,
                    gn_w, gn_b, gn_ms)
    return out[:N]


# speedup vs baseline: 18.4765x; 1.2076x over previous
import functools

import jax
import jax.numpy as jnp
from jax import lax
from jax.experimental import pallas as pl
from jax.experimental.pallas import tpu as pltpu
from jax.experimental.pallas import tpu_sc as plsc

N = 10000
E = 320000
D = 128
NC = 2
NS = 16
NW = NC * NS
NP = 10240
STR = NP // NS
EPT = E // NW
CH = 80
NCH = EPT // CH
BR = 512
NB = NP // BR
HN = NP // 2
ACC_R = HN + 128
OSTR = HN // NS
ASTR = ACC_R // NS
DUMMY = HN
EPTP = EPT + 96
PSH = 13
ECH = 400
NEC = EPT // ECH
F32 = jnp.float32
I32 = jnp.int32


def _mesh():
    return plsc.VectorSubcoreMesh(
        core_axis_name="c", subcore_axis_name="s", num_cores=NC, num_subcores=NS)


def _seg_combine(kbuf, vbuf, k16, v16, op):
    ks, vs = plsc.sort_key_val(k16, v16)
    iota = lax.iota(I32, 16)
    kbuf[...] = ks
    val = vs
    for s in (1, 2, 4, 8):
        vbuf[...] = val
        idx = jnp.maximum(iota - s, 0)
        kp = plsc.load_gather(kbuf, [idx])
        vp = plsc.load_gather(vbuf, [idx])
        same = (kp == ks) & (iota >= s)
        val = jnp.where(same, op(val, vp), val)
    kn = plsc.load_gather(kbuf, [jnp.minimum(iota + 1, 15)])
    endmask = (kn != ks) | (iota == 15)
    return ks, val, endmask


def _combine_tiles(loc, shared, stripebuf, accb, out_ref, cid, sid, op):
    pltpu.sync_copy(loc, shared.at[sid])
    plsc.subcore_barrier()
    base = sid * STR
    pltpu.sync_copy(shared.at[:, pl.ds(base, STR)], stripebuf)

    def body(i, _):
        v = stripebuf[0, pl.ds(i * 16, 16)]
        for k in range(1, NS):
            v = op(v, stripebuf[k, pl.ds(i * 16, 16)])
        accb[pl.ds(i * 16, 16)] = v
        return 0

    lax.fori_loop(0, STR // 16, body, 0)
    pltpu.sync_copy(accb, out_ref.at[cid, pl.ds(base, STR)])


def _fill_np(loc, value):
    v16 = jnp.full((16,), value, F32)

    def body(i, _):
        loc[pl.ds(i * 16, 16)] = v16
        return 0

    lax.fori_loop(0, NP // 16, body, 0)


def _zero_rows(rows):
    z16 = jnp.zeros((16,), F32)

    def body(r, _):
        for c in range(D // 16):
            rows[r, pl.ds(c * 16, 16)] = z16
        return 0

    lax.fori_loop(0, CH, body, 0)


def _zero_acc_stripe(rows, acc_sh, sid):
    _zero_rows(rows)
    base = sid * ASTR
    for k in range(ASTR // CH):
        pltpu.sync_copy(rows, acc_sh.at[pl.ds(base + k * CH, CH), :])
    rem = ASTR % CH
    if rem:
        pltpu.sync_copy(rows.at[pl.ds(0, rem), :],
                        acc_sh.at[pl.ds(base + (ASTR // CH) * CH, rem), :])


def _scale_rows(rows, get_scale):
    def body(r, _):
        b = get_scale(r)
        for c in range(D // 16):
            rows[r, pl.ds(c * 16, 16)] = rows[r, pl.ds(c * 16, 16)] * b
        return 0

    lax.fori_loop(0, CH, body, 0)


def _compact_store(cbuf, pendc, half, cnt, pend, d16, s16,
                   sbuf=None, pends=None, sc16=None):
    rel = d16 - half * HN
    ok = (rel >= 0) & (rel < HN)
    p = (s16 << PSH) | jnp.where(ok, rel, 0)
    plsc.store_compressed(pendc.at[half, pl.ds(pend, 16)], p, mask=ok)
    if sbuf is not None:
        plsc.store_compressed(pends.at[half, pl.ds(pend, 16)], sc16, mask=ok)
    ntot = pend + jnp.max(plsc.all_reduce_population_count(ok))
    flush = ntot >= 16

    @pl.when(flush)
    def _():
        cbuf[half, pl.ds(cnt, 16)] = pendc[half, pl.ds(0, 16)]
        pendc[half, pl.ds(0, 16)] = pendc[half, pl.ds(16, 16)]
        if sbuf is not None:
            sbuf[half, pl.ds(cnt, 16)] = pends[half, pl.ds(0, 16)]
            pends[half, pl.ds(0, 16)] = pends[half, pl.ds(16, 16)]

    return jnp.where(flush, cnt + 16, cnt), jnp.where(flush, ntot - 16, ntot)


def _finalize_compacted(cbuf, pendc, half, cnt, pend, sbuf=None, pends=None):
    iota = lax.iota(I32, 16)
    v = jnp.where(iota < pend, pendc[half, pl.ds(0, 16)], DUMMY)
    cbuf[half, pl.ds(cnt, 16)] = v
    if sbuf is not None:
        sv = jnp.where(iota < pend, pends[half, pl.ds(0, 16)], 0.0)
        sbuf[half, pl.ds(cnt, 16)] = sv
    total = cnt + 16
    target = ((total + CH - 1) // CH) * CH
    padp = jnp.full((16,), DUMMY, I32)
    padz = jnp.zeros((16,), F32)

    def body(i, _):
        cbuf[half, pl.ds(total + i * 16, 16)] = padp
        if sbuf is not None:
            sbuf[half, pl.ds(total + i * 16, 16)] = padz
        return 0

    lax.fori_loop(0, (target - total) // 16, body, 0)
    return target // CH


def _unpack_chunk(cbuf, half, base, srcb, dstb, g):
    p16 = cbuf[half, pl.ds(base + g * 16, 16)]
    s16 = p16 >> PSH
    r16 = p16 & ((1 << PSH) - 1)
    srcb[pl.ds(g * 16, 16)] = s16
    dstb[pl.ds(g * 16, 16)] = r16
    return s16, r16


def _copy_out_stripe(out_hbm, acc_sh, cid, half, sid):
    pltpu.sync_copy(acc_sh.at[pl.ds(sid * OSTR, OSTR), :],
                    out_hbm.at[cid, half, pl.ds(sid * OSTR, OSTR), :])


def _stream_half_db(h_hbm, out_hbm, acc_sh, rows2, srcb2, dstb2, sem2,
                    half, nch, cid, sid, prep, scale_fn):
    def issue(idx, b):
        prep(idx, b)
        pltpu.async_copy(h_hbm.at[srcb2[b]], rows2[b], sem2[b])

    def consume(idx, b):
        pltpu.make_async_copy(h_hbm.at[srcb2[b]], rows2[b], sem2[b]).wait()
        _scale_rows(rows2[b], lambda r: scale_fn(idx, b, r))
        pltpu.sync_copy(rows2[b], acc_sh.at[dstb2[b]], add=True)

    _zero_acc_stripe(rows2[0], acc_sh, sid)
    plsc.subcore_barrier()

    @pl.when(nch > 0)
    def _():
        issue(0, 0)

    def body(j, _):
        for b in (0, 1):
            idx = j * 2 + b

            @pl.when(idx < nch)
            def _(idx=idx, b=b):
                @pl.when(idx + 1 < nch)
                def _():
                    issue(idx + 1, 1 - b)

                consume(idx, b)

        return 0

    lax.fori_loop(0, (nch + 1) // 2, body, 0)
    plsc.subcore_barrier()
    _copy_out_stripe(out_hbm, acc_sh, cid, half, sid)


def _deg_body(dst_hbm, w_hbm, out_hbm, dstv, wv, loc, kbuf, vbuf,
              shared, stripebuf, accb):
    cid = lax.axis_index("c")
    sid = lax.axis_index("s")
    wid = cid * NS + sid
    pltpu.sync_copy(dst_hbm.at[wid], dstv)
    pltpu.sync_copy(w_hbm.at[wid], wv)
    _fill_np(loc, 0.0)

    def body(j, _):
        for g in range(CH // 16):
            d16 = dstv[j, pl.ds(g * 16, 16)]
            v16 = wv[j, pl.ds(g * 16, 16)]
            ks, tot, em = _seg_combine(kbuf, vbuf, d16, v16, lambda a, b: a + b)
            plsc.addupdate_scatter(loc, [ks], tot, mask=em)
        return 0

    lax.fori_loop(0, NCH, body, 0)
    _combine_tiles(loc, shared, stripebuf, accb, out_hbm, cid, sid,
                   lambda a, b: a + b)


def _deg_call(dst_r, w_r):
    k = functools.partial(
        pl.kernel,
        out_type=jax.ShapeDtypeStruct((NC, NP), F32),
        mesh=_mesh(),
        compiler_params=pltpu.CompilerParams(needs_layout_passes=False),
        scratch_types=[
            pltpu.VMEM((NCH, CH), I32),
            pltpu.VMEM((NCH, CH), F32),
            pltpu.VMEM((NP,), F32),
            pltpu.VMEM((16,), I32),
            pltpu.VMEM((16,), F32),
            pltpu.VMEM_SHARED((NS, NP), F32),
            pltpu.VMEM((NS, STR), F32),
            pltpu.VMEM((STR,), F32),
        ],
    )(_deg_body)
    return k(dst_r, w_r)


def _gcn_body(src_hbm, dst_hbm, w_hbm, dis_hbm, h_hbm, out_hbm,
              srcc, dstc, wc, disv, cbuf, sbuf, pendc, pends,
              rowsA, rowsB, srcbA, srcbB, dstbA, dstbB, semA, semB, acc_sh):
    cid = lax.axis_index("c")
    sid = lax.axis_index("s")
    wid = cid * NS + sid
    pltpu.sync_copy(dis_hbm, disv)

    def outer(r, carry):
        base = wid * EPT + r * ECH
        pltpu.sync_copy(src_hbm.at[pl.ds(base, ECH)], srcc)
        pltpu.sync_copy(dst_hbm.at[pl.ds(base, ECH)], dstc)
        pltpu.sync_copy(w_hbm.at[pl.ds(base, ECH)], wc)

        def cb(g, carry):
            c0, p0, c1, p1 = carry
            s16 = srcc[pl.ds(g * 16, 16)]
            d16 = dstc[pl.ds(g * 16, 16)]
            sc = wc[pl.ds(g * 16, 16)] * plsc.load_gather(disv, [s16])
            c0, p0 = _compact_store(cbuf, pendc, 0, c0, p0, d16, s16,
                                    sbuf, pends, sc)
            c1, p1 = _compact_store(cbuf, pendc, 1, c1, p1, d16, s16,
                                    sbuf, pends, sc)
            return c0, p0, c1, p1

        return lax.fori_loop(0, ECH // 16, cb, carry)

    z = jnp.int32(0)
    c0, p0, c1, p1 = lax.fori_loop(0, NEC, outer, (z, z, z, z))
    rows2, srcb2, dstb2, sem2 = ((rowsA, rowsB), (srcbA, srcbB),
                                 (dstbA, dstbB), (semA, semB))
    for half, nch in ((0, _finalize_compacted(cbuf, pendc, 0, c0, p0,
                                              sbuf, pends)),
                      (1, _finalize_compacted(cbuf, pendc, 1, c1, p1,
                                              sbuf, pends))):
        h16 = jnp.full((16,), half, I32)

        def prep(idx, b):
            for g in range(CH // 16):
                _unpack_chunk(cbuf, half, idx * CH, srcb2[b], dstb2[b], g)

        def scale_fn(idx, b, r):
            return plsc.load_gather(
                sbuf, [h16, jnp.broadcast_to(idx * CH + r, (16,))])

        _stream_half_db(h_hbm, out_hbm, acc_sh, rows2, srcb2, dstb2, sem2,
                        half, nch, cid, sid, prep, scale_fn)


def _gcn_call(src, dst, w, dis, h):
    k = functools.partial(
        pl.kernel,
        out_type=jax.ShapeDtypeStruct((NC, 2, HN, D), F32),
        mesh=_mesh(),
        compiler_params=pltpu.CompilerParams(needs_layout_passes=False),
        scratch_types=[
            pltpu.VMEM((ECH,), I32),
            pltpu.VMEM((ECH,), I32),
            pltpu.VMEM((ECH,), F32),
            pltpu.VMEM((NP,), F32),
            pltpu.VMEM((2, EPTP), I32),
            pltpu.VMEM((2, EPTP), F32),
            pltpu.VMEM((2, 48), I32),
            pltpu.VMEM((2, 48), F32),
            pltpu.VMEM((CH, D), F32),
            pltpu.VMEM((CH, D), F32),
            pltpu.VMEM((CH,), I32),
            pltpu.VMEM((CH,), I32),
            pltpu.VMEM((CH,), I32),
            pltpu.VMEM((CH,), I32),
            pltpu.SemaphoreType.DMA,
            pltpu.SemaphoreType.DMA,
            pltpu.VMEM_SHARED((ACC_R, D), F32),
        ],
    )(_gcn_body)
    return k(src, dst, w, dis, h)


def _amax_body(src_hbm, dst_hbm, asrc_hbm, adst_hbm, out_hbm,
               srcv, dstv, asv, adv, loc, kbuf, vbuf, shared, stripebuf, accb):
    cid = lax.axis_index("c")
    sid = lax.axis_index("s")
    wid = cid * NS + sid
    pltpu.sync_copy(src_hbm.at[wid], srcv)
    pltpu.sync_copy(dst_hbm.at[wid], dstv)
    pltpu.sync_copy(asrc_hbm, asv)
    pltpu.sync_copy(adst_hbm, adv)
    _fill_np(loc, -1e30)

    def body(j, _):
        for g in range(CH // 16):
            s16 = srcv[j, pl.ds(g * 16, 16)]
            d16 = dstv[j, pl.ds(g * 16, 16)]
            a = plsc.load_gather(asv, [s16]) + plsc.load_gather(adv, [d16])
            a = jnp.where(a > 0, a, 0.2 * a)
            ks, tot, em = _seg_combine(kbuf, vbuf, d16, a, jnp.maximum)
            cur = plsc.load_gather(loc, [ks])
            plsc.store_scatter(loc, [ks], jnp.maximum(cur, tot), mask=em)
        return 0

    lax.fori_loop(0, NCH, body, 0)
    _combine_tiles(loc, shared, stripebuf, accb, out_hbm, cid, sid, jnp.maximum)


def _amax_call(src_r, dst_r, asrc, adst):
    k = functools.partial(
        pl.kernel,
        out_type=jax.ShapeDtypeStruct((NC, NP), F32),
        mesh=_mesh(),
        compiler_params=pltpu.CompilerParams(needs_layout_passes=False),
        scratch_types=[
            pltpu.VMEM((NCH, CH), I32),
            pltpu.VMEM((NCH, CH), I32),
            pltpu.VMEM((NP,), F32),
            pltpu.VMEM((NP,), F32),
            pltpu.VMEM((NP,), F32),
            pltpu.VMEM((16,), I32),
            pltpu.VMEM((16,), F32),
            pltpu.VMEM_SHARED((NS, NP), F32),
            pltpu.VMEM((NS, STR), F32),
            pltpu.VMEM((STR,), F32),
        ],
    )(_amax_body)
    return k(src_r, dst_r, asrc, adst)


def _gat_body(src_hbm, dst_hbm, asrc_hbm, adst_hbm, amax_hbm, h2_hbm,
              outacc_hbm, outden_hbm,
              srcc, dstc, asv, adv, amv, cbuf, pendc,
              rowsA, rowsB, srcbA, srcbB, dstbA, dstbB, scalebA, scalebB,
              denloc, kbuf, vbuf, semA, semB, acc_sh):
    cid = lax.axis_index("c")
    sid = lax.axis_index("s")
    wid = cid * NS + sid
    pltpu.sync_copy(asrc_hbm, asv)
    pltpu.sync_copy(adst_hbm, adv)
    pltpu.sync_copy(amax_hbm, amv)
    _fill_np(denloc, 0.0)

    def outer(r, carry):
        base = wid * EPT + r * ECH
        pltpu.sync_copy(src_hbm.at[pl.ds(base, ECH)], srcc)
        pltpu.sync_copy(dst_hbm.at[pl.ds(base, ECH)], dstc)

        def cb(g, carry):
            c0, p0, c1, p1 = carry
            s16 = srcc[pl.ds(g * 16, 16)]
            d16 = dstc[pl.ds(g * 16, 16)]
            c0, p0 = _compact_store(cbuf, pendc, 0, c0, p0, d16, s16)
            c1, p1 = _compact_store(cbuf, pendc, 1, c1, p1, d16, s16)
            return c0, p0, c1, p1

        return lax.fori_loop(0, ECH // 16, cb, carry)

    z = jnp.int32(0)
    c0, p0, c1, p1 = lax.fori_loop(0, NEC, outer, (z, z, z, z))
    rows2, srcb2, dstb2, sem2 = ((rowsA, rowsB), (srcbA, srcbB),
                                 (dstbA, dstbB), (semA, semB))
    scaleb2 = (scalebA, scalebB)
    for half, nch in ((0, _finalize_compacted(cbuf, pendc, 0, c0, p0)),
                      (1, _finalize_compacted(cbuf, pendc, 1, c1, p1))):

        def prep(idx, b):
            for g in range(CH // 16):
                s16, r16 = _unpack_chunk(cbuf, half, idx * CH,
                                         srcb2[b], dstb2[b], g)
                pad = r16 == DUMMY
                dabs = jnp.where(pad, 0, r16 + half * HN)
                a = (plsc.load_gather(asv, [s16])
                     + plsc.load_gather(adv, [dabs]))
                a = jnp.where(a > 0, a, 0.2 * a)
                e = jnp.exp(a - plsc.load_gather(amv, [dabs]))
                e = jnp.where(pad, 0.0, e)
                scaleb2[b][pl.ds(g * 16, 16)] = e
                ks, tot, em = _seg_combine(kbuf, vbuf, dabs, e,
                                           lambda x, y: x + y)
                plsc.addupdate_scatter(denloc, [ks], tot, mask=em)

        def scale_fn(idx, b, r):
            return plsc.load_gather(scaleb2[b], [jnp.broadcast_to(r, (16,))])

        _stream_half_db(h2_hbm, outacc_hbm, acc_sh, rows2, srcb2, dstb2,
                        sem2, half, nch, cid, sid, prep, scale_fn)
    pltpu.sync_copy(denloc, outden_hbm.at[cid, sid])


def _gat_call(src, dst, asrc, adst, amax, h2):
    k = functools.partial(
        pl.kernel,
        out_type=[jax.ShapeDtypeStruct((NC, 2, HN, D), F32),
                  jax.ShapeDtypeStruct((NC, NS, NP), F32)],
        mesh=_mesh(),
        compiler_params=pltpu.CompilerParams(needs_layout_passes=False),
        scratch_types=[
            pltpu.VMEM((ECH,), I32),
            pltpu.VMEM((ECH,), I32),
            pltpu.VMEM((NP,), F32),
            pltpu.VMEM((NP,), F32),
            pltpu.VMEM((NP,), F32),
            pltpu.VMEM((2, EPTP), I32),
            pltpu.VMEM((2, 48), I32),
            pltpu.VMEM((CH, D), F32),
            pltpu.VMEM((CH, D), F32),
            pltpu.VMEM((CH,), I32),
            pltpu.VMEM((CH,), I32),
            pltpu.VMEM((CH,), I32),
            pltpu.VMEM((CH,), I32),
            pltpu.VMEM((CH,), F32),
            pltpu.VMEM((CH,), F32),
            pltpu.VMEM((NP,), F32),
            pltpu.VMEM((16,), I32),
            pltpu.VMEM((16,), F32),
            pltpu.SemaphoreType.DMA,
            pltpu.SemaphoreType.DMA,
            pltpu.VMEM_SHARED((ACC_R, D), F32),
        ],
    )(_gat_body)
    return k(src, dst, asrc, adst, amax, h2)


def _mm_body(x_ref, w_ref, o_ref):
    o_ref[...] = jnp.dot(x_ref[...], w_ref[...],
                         preferred_element_type=jnp.float32)


def _matmul(xp, W):
    return pl.pallas_call(
        _mm_body,
        grid=(NB,),
        in_specs=[pl.BlockSpec((BR, D), lambda i: (i, 0)),
                  pl.BlockSpec((D, D), lambda i: (0, 0))],
        out_specs=pl.BlockSpec((BR, D), lambda i: (i, 0)),
        out_shape=jax.ShapeDtypeStruct((NP, D), F32),
    )(xp, W)


def _tc2_body(x_ref, h_ref, a0_ref, a1_ref, dis_ref,
              b1_ref, gnw_ref, gnb_ref, gnms_ref, Wg_ref, as_ref, ad_ref,
              x1_ref, h2_ref, asv_ref, adv_ref, sum_ref, sq_ref):
    p = pl.program_id(0)
    i = pl.program_id(1)
    disv = dis_ref[...][:, None]
    acc = a0_ref[...] + a1_ref[...]
    t = disv * (acc + disv * h_ref[...]) + b1_ref[...][None, :]
    rid = i * BR + lax.broadcasted_iota(I32, (BR, 1), 0)
    t = jnp.where(rid < N, t, 0.0)

    @pl.when(p == 0)
    def _():
        @pl.when(i == 0)
        def _():
            sum_ref[...] = jnp.zeros_like(sum_ref)
            sq_ref[...] = jnp.zeros_like(sq_ref)

        sum_ref[...] += jnp.sum(t, axis=0, keepdims=True)
        sq_ref[...] += jnp.sum(t * t, axis=0, keepdims=True)

    @pl.when(p == 1)
    def _():
        ms = gnms_ref[...][None, :]
        mean = sum_ref[...] / N
        e2 = sq_ref[...] / N
        var = e2 - (2.0 - ms) * ms * mean * mean
        gn = gnw_ref[...][None, :] * (t - ms * mean) / jnp.sqrt(var + 1e-5) \
            + gnb_ref[...][None, :]
        l = jnp.where(gn > 0, gn, 0.01 * gn)
        x1 = x_ref[...] + l
        x1_ref[...] = x1
        h2 = jnp.dot(x1, Wg_ref[...], preferred_element_type=jnp.float32)
        h2_ref[...] = h2
        asv_ref[...] = jnp.sum(h2 * as_ref[...][None, :], axis=1)
        adv_ref[...] = jnp.sum(h2 * ad_ref[...][None, :], axis=1)


def _tc2_call(xp, h, a0, a1, dis, b1, gn_w, gn_b, gn_ms, Wg,
              att_src, att_dst):
    mat = lambda: pl.BlockSpec((BR, D), lambda p, i: (i, 0))
    vec = lambda: pl.BlockSpec((BR,), lambda p, i: (i,))
    dvec = lambda: pl.BlockSpec((D,), lambda p, i: (0,))
    return pl.pallas_call(
        _tc2_body,
        grid=(2, NB),
        in_specs=[mat(), mat(), mat(), mat(), vec(),
                  dvec(), dvec(), dvec(), dvec(),
                  pl.BlockSpec((D, D), lambda p, i: (0, 0)), dvec(), dvec()],
        out_specs=[mat(), mat(), vec(), vec()],
        out_shape=[jax.ShapeDtypeStruct((NP, D), F32),
                   jax.ShapeDtypeStruct((NP, D), F32),
                   jax.ShapeDtypeStruct((NP,), F32),
                   jax.ShapeDtypeStruct((NP,), F32)],
        scratch_shapes=[pltpu.VMEM((1, D), F32), pltpu.VMEM((1, D), F32)],
    )(xp, h, a0, a1, dis, b1, gn_w, gn_b, gn_ms, Wg, att_src, att_dst)


def _tc3_body(x1_ref, h2_ref, g0_ref, g1_ref, es_ref,
              den_ref, bg_ref, gnw_ref, gnb_ref, gnms_ref, out_ref,
              sum_ref, sq_ref):
    p = pl.program_id(0)
    i = pl.program_id(1)
    esv = es_ref[...][:, None]
    denv = (jnp.sum(den_ref[...], axis=0) + es_ref[...])[:, None]
    acc = g0_ref[...] + g1_ref[...]
    t = (acc + h2_ref[...] * esv) / (denv + 1e-16) + bg_ref[...][None, :]
    rid = i * BR + lax.broadcasted_iota(I32, (BR, 1), 0)
    t = jnp.where(rid < N, t, 0.0)

    @pl.when(p == 0)
    def _():
        @pl.when(i == 0)
        def _():
            sum_ref[...] = jnp.zeros_like(sum_ref)
            sq_ref[...] = jnp.zeros_like(sq_ref)

        sum_ref[...] += jnp.sum(t, axis=0, keepdims=True)
        sq_ref[...] += jnp.sum(t * t, axis=0, keepdims=True)

    @pl.when(p == 1)
    def _():
        ms = gnms_ref[...][None, :]
        mean = sum_ref[...] / N
        e2 = sq_ref[...] / N
        var = e2 - (2.0 - ms) * ms * mean * mean
        gn = gnw_ref[...][None, :] * (t - ms * mean) / jnp.sqrt(var + 1e-5) \
            + gnb_ref[...][None, :]
        l = jnp.where(gn > 0, gn, 0.01 * gn)
        out_ref[...] = x1_ref[...] + l


def _tc3_call(x1, h2, g0, g1, es, den, bg, gn_w, gn_b, gn_ms):
    mat = lambda: pl.BlockSpec((BR, D), lambda p, i: (i, 0))
    vec = lambda: pl.BlockSpec((BR,), lambda p, i: (i,))
    dvec = lambda: pl.BlockSpec((D,), lambda p, i: (0,))
    return pl.pallas_call(
        _tc3_body,
        grid=(2, NB),
        in_specs=[mat(), mat(), mat(), mat(), vec(),
                  pl.BlockSpec((NW, BR), lambda p, i: (0, i)),
                  dvec(), dvec(), dvec(), dvec()],
        out_specs=mat(),
        out_shape=jax.ShapeDtypeStruct((NP, D), F32),
        scratch_shapes=[pltpu.VMEM((1, D), F32), pltpu.VMEM((1, D), F32)],
    )(x1, h2, g0, g1, es, den, bg, gn_w, gn_b, gn_ms)


def kernel(x, edges, weight, W1, b1, gn_w, gn_b, gn_ms, Wg, att_src, att_dst,
           bg):
    xp = jnp.zeros((NP, D), F32).at[:N].set(x)
    src_r = edges[0].reshape(NW, NCH, CH)
    dst_r = edges[1].reshape(NW, NCH, CH)
    w_r = weight.reshape(NW, NCH, CH)

    h = _matmul(xp, W1)
    degp = _deg_call(dst_r, w_r)
    deg = degp[0] + degp[1] + 1.0
    dis = jnp.where(deg > 0, 1.0 / jnp.sqrt(deg), 0.0)
    accA = _gcn_call(edges[0], edges[1], weight, dis, h).reshape(NC, NP, D)
    x1, h2, asv, adv = _tc2_call(xp, h, accA[0], accA[1], dis, b1,
                                 gn_w, gn_b, gn_ms, Wg, att_src, att_dst)

    amaxp = _amax_call(src_r, dst_r, asv, adv)
    aself = asv + adv
    aself = jnp.where(aself > 0, aself, 0.2 * aself)
    amax = jnp.maximum(jnp.maximum(amaxp[0], amaxp[1]), aself)
    accG, denp = _gat_call(edges[0], edges[1], asv, adv, amax, h2)
    accG = accG.reshape(NC, NP, D)
    es = jnp.exp(aself - amax)
    out = _tc3_call(x1, h2, accG[0], accG[1], es, denp.reshape(NW, NP), bg,
                    gn_w, gn_b, gn_ms)
    return out[:N]


# speedup vs baseline: 18.7673x; 1.0157x over previous
import functools

import jax
import jax.numpy as jnp
from jax import lax
from jax.experimental import pallas as pl
from jax.experimental.pallas import tpu as pltpu
from jax.experimental.pallas import tpu_sc as plsc

N = 10000
E = 320000
D = 128
NC = 2
NS = 16
NW = NC * NS
NP = 10240
STR = NP // NS
EPT = E // NW
CH = 80
NCH = EPT // CH
BR = 512
NB = NP // BR
HN = NP // 2
ACC_R = HN + 128
OSTR = HN // NS
ASTR = ACC_R // NS
DUMMY = HN
EPTP = EPT + 96
PSH = 13
ECH = 400
NEC = EPT // ECH
F32 = jnp.float32
I32 = jnp.int32


def _mesh():
    return plsc.VectorSubcoreMesh(
        core_axis_name="c", subcore_axis_name="s", num_cores=NC, num_subcores=NS)


def _seg_combine(kbuf, vbuf, k16, v16, op):
    ks, vs = plsc.sort_key_val(k16, v16)
    iota = lax.iota(I32, 16)
    kbuf[...] = ks
    val = vs
    for s in (1, 2, 4, 8):
        vbuf[...] = val
        idx = jnp.maximum(iota - s, 0)
        kp = plsc.load_gather(kbuf, [idx])
        vp = plsc.load_gather(vbuf, [idx])
        same = (kp == ks) & (iota >= s)
        val = jnp.where(same, op(val, vp), val)
    kn = plsc.load_gather(kbuf, [jnp.minimum(iota + 1, 15)])
    endmask = (kn != ks) | (iota == 15)
    return ks, val, endmask


def _combine_tiles(loc, shared, stripebuf, accb, out_ref, cid, sid, op):
    pltpu.sync_copy(loc, shared.at[sid])
    plsc.subcore_barrier()
    base = sid * STR
    pltpu.sync_copy(shared.at[:, pl.ds(base, STR)], stripebuf)

    def body(i, _):
        v = stripebuf[0, pl.ds(i * 16, 16)]
        for k in range(1, NS):
            v = op(v, stripebuf[k, pl.ds(i * 16, 16)])
        accb[pl.ds(i * 16, 16)] = v
        return 0

    lax.fori_loop(0, STR // 16, body, 0)
    pltpu.sync_copy(accb, out_ref.at[cid, pl.ds(base, STR)])


def _fill_np(loc, value):
    v16 = jnp.full((16,), value, F32)

    def body(i, _):
        loc[pl.ds(i * 16, 16)] = v16
        return 0

    lax.fori_loop(0, NP // 16, body, 0)


def _zero_rows(rows):
    z16 = jnp.zeros((16,), F32)

    def body(r, _):
        for c in range(D // 16):
            rows[r, pl.ds(c * 16, 16)] = z16
        return 0

    lax.fori_loop(0, CH, body, 0)


def _zero_acc_stripe(rows, acc_sh, sid):
    _zero_rows(rows)
    base = sid * ASTR
    for k in range(ASTR // CH):
        pltpu.sync_copy(rows, acc_sh.at[pl.ds(base + k * CH, CH), :])
    rem = ASTR % CH
    if rem:
        pltpu.sync_copy(rows.at[pl.ds(0, rem), :],
                        acc_sh.at[pl.ds(base + (ASTR // CH) * CH, rem), :])


def _scale_rows(rows, get_scale):
    def body(r2, _):
        for rr in range(2):
            r = r2 * 2 + rr
            b = get_scale(r)
            for c in range(D // 16):
                rows[r, pl.ds(c * 16, 16)] = rows[r, pl.ds(c * 16, 16)] * b
        return 0

    lax.fori_loop(0, CH // 2, body, 0)


def _compact_store(cbuf, pendc, half, cnt, pend, d16, s16,
                   sbuf=None, pends=None, sc16=None):
    rel = d16 - half * HN
    ok = (rel >= 0) & (rel < HN)
    p = (s16 << PSH) | jnp.where(ok, rel, 0)
    plsc.store_compressed(pendc.at[half, pl.ds(pend, 16)], p, mask=ok)
    if sbuf is not None:
        plsc.store_compressed(pends.at[half, pl.ds(pend, 16)], sc16, mask=ok)
    ntot = pend + jnp.max(plsc.all_reduce_population_count(ok))
    flush = ntot >= 16

    @pl.when(flush)
    def _():
        cbuf[half, pl.ds(cnt, 16)] = pendc[half, pl.ds(0, 16)]
        pendc[half, pl.ds(0, 16)] = pendc[half, pl.ds(16, 16)]
        if sbuf is not None:
            sbuf[half, pl.ds(cnt, 16)] = pends[half, pl.ds(0, 16)]
            pends[half, pl.ds(0, 16)] = pends[half, pl.ds(16, 16)]

    return jnp.where(flush, cnt + 16, cnt), jnp.where(flush, ntot - 16, ntot)


def _finalize_compacted(cbuf, pendc, half, cnt, pend, sbuf=None, pends=None):
    iota = lax.iota(I32, 16)
    v = jnp.where(iota < pend, pendc[half, pl.ds(0, 16)], DUMMY)
    cbuf[half, pl.ds(cnt, 16)] = v
    if sbuf is not None:
        sv = jnp.where(iota < pend, pends[half, pl.ds(0, 16)], 0.0)
        sbuf[half, pl.ds(cnt, 16)] = sv
    total = cnt + 16
    target = ((total + CH - 1) // CH) * CH
    padp = jnp.full((16,), DUMMY, I32)
    padz = jnp.zeros((16,), F32)

    def body(i, _):
        cbuf[half, pl.ds(total + i * 16, 16)] = padp
        if sbuf is not None:
            sbuf[half, pl.ds(total + i * 16, 16)] = padz
        return 0

    lax.fori_loop(0, (target - total) // 16, body, 0)
    return target // CH


def _unpack_chunk(cbuf, half, base, srcb, dstb, g):
    p16 = cbuf[half, pl.ds(base + g * 16, 16)]
    s16 = p16 >> PSH
    r16 = p16 & ((1 << PSH) - 1)
    srcb[pl.ds(g * 16, 16)] = s16
    dstb[pl.ds(g * 16, 16)] = r16
    return s16, r16


def _copy_out_stripe(out_hbm, acc_sh, cid, half, sid):
    pltpu.sync_copy(acc_sh.at[pl.ds(sid * OSTR, OSTR), :],
                    out_hbm.at[cid, half, pl.ds(sid * OSTR, OSTR), :])


def _stream_half_db(h_hbm, out_hbm, acc_sh, rows2, srcb2, dstb2, sem2, ssem2,
                    half, nch, cid, sid, prep, scale_fn):
    def wait_scatter(b):
        pltpu.make_async_copy(rows2[b], acc_sh.at[dstb2[b]],
                              ssem2[b]).wait()

    def issue(idx, b):
        @pl.when(idx >= 2)
        def _():
            wait_scatter(b)

        prep(idx, b)
        pltpu.async_copy(h_hbm.at[srcb2[b]], rows2[b], sem2[b])

    def consume(idx, b):
        pltpu.make_async_copy(h_hbm.at[srcb2[b]], rows2[b], sem2[b]).wait()
        _scale_rows(rows2[b], lambda r: scale_fn(idx, b, r))
        pltpu.async_copy(rows2[b], acc_sh.at[dstb2[b]], ssem2[b], add=True)

    _zero_acc_stripe(rows2[0], acc_sh, sid)
    plsc.subcore_barrier()

    @pl.when(nch > 0)
    def _():
        issue(0, 0)

    def body(j, _):
        for b in (0, 1):
            idx = j * 2 + b

            @pl.when(idx < nch)
            def _(idx=idx, b=b):
                @pl.when(idx + 1 < nch)
                def _():
                    issue(idx + 1, 1 - b)

                consume(idx, b)

        return 0

    lax.fori_loop(0, (nch + 1) // 2, body, 0)

    @pl.when(nch >= 1)
    def _():
        wait_scatter(0)

    @pl.when(nch >= 2)
    def _():
        wait_scatter(1)

    plsc.subcore_barrier()
    _copy_out_stripe(out_hbm, acc_sh, cid, half, sid)


def _deg_body(dst_hbm, w_hbm, out_hbm, dstv, wv, loc, kbuf, vbuf,
              shared, stripebuf, accb):
    cid = lax.axis_index("c")
    sid = lax.axis_index("s")
    wid = cid * NS + sid
    pltpu.sync_copy(dst_hbm.at[wid], dstv)
    pltpu.sync_copy(w_hbm.at[wid], wv)
    _fill_np(loc, 0.0)

    def body(j, _):
        for g in range(CH // 16):
            d16 = dstv[j, pl.ds(g * 16, 16)]
            v16 = wv[j, pl.ds(g * 16, 16)]
            ks, tot, em = _seg_combine(kbuf, vbuf, d16, v16, lambda a, b: a + b)
            plsc.addupdate_scatter(loc, [ks], tot, mask=em)
        return 0

    lax.fori_loop(0, NCH, body, 0)
    _combine_tiles(loc, shared, stripebuf, accb, out_hbm, cid, sid,
                   lambda a, b: a + b)


def _deg_call(dst_r, w_r):
    k = functools.partial(
        pl.kernel,
        out_type=jax.ShapeDtypeStruct((NC, NP), F32),
        mesh=_mesh(),
        compiler_params=pltpu.CompilerParams(needs_layout_passes=False),
        scratch_types=[
            pltpu.VMEM((NCH, CH), I32),
            pltpu.VMEM((NCH, CH), F32),
            pltpu.VMEM((NP,), F32),
            pltpu.VMEM((16,), I32),
            pltpu.VMEM((16,), F32),
            pltpu.VMEM_SHARED((NS, NP), F32),
            pltpu.VMEM((NS, STR), F32),
            pltpu.VMEM((STR,), F32),
        ],
    )(_deg_body)
    return k(dst_r, w_r)


def _gcn_body(src_hbm, dst_hbm, w_hbm, dis_hbm, h_hbm, out_hbm,
              srcc, dstc, wc, disv, cbuf, sbuf, pendc, pends,
              rowsA, rowsB, srcbA, srcbB, dstbA, dstbB, semA, semB,
              ssemA, ssemB, acc_sh):
    cid = lax.axis_index("c")
    sid = lax.axis_index("s")
    wid = cid * NS + sid
    pltpu.sync_copy(dis_hbm, disv)

    def outer(r, carry):
        base = wid * EPT + r * ECH
        pltpu.sync_copy(src_hbm.at[pl.ds(base, ECH)], srcc)
        pltpu.sync_copy(dst_hbm.at[pl.ds(base, ECH)], dstc)
        pltpu.sync_copy(w_hbm.at[pl.ds(base, ECH)], wc)

        def cb(g, carry):
            c0, p0, c1, p1 = carry
            s16 = srcc[pl.ds(g * 16, 16)]
            d16 = dstc[pl.ds(g * 16, 16)]
            sc = wc[pl.ds(g * 16, 16)] * plsc.load_gather(disv, [s16])
            c0, p0 = _compact_store(cbuf, pendc, 0, c0, p0, d16, s16,
                                    sbuf, pends, sc)
            c1, p1 = _compact_store(cbuf, pendc, 1, c1, p1, d16, s16,
                                    sbuf, pends, sc)
            return c0, p0, c1, p1

        return lax.fori_loop(0, ECH // 16, cb, carry)

    z = jnp.int32(0)
    c0, p0, c1, p1 = lax.fori_loop(0, NEC, outer, (z, z, z, z))
    rows2, srcb2, dstb2, sem2 = ((rowsA, rowsB), (srcbA, srcbB),
                                 (dstbA, dstbB), (semA, semB))
    ssem2 = (ssemA, ssemB)
    for half, nch in ((0, _finalize_compacted(cbuf, pendc, 0, c0, p0,
                                              sbuf, pends)),
                      (1, _finalize_compacted(cbuf, pendc, 1, c1, p1,
                                              sbuf, pends))):
        h16 = jnp.full((16,), half, I32)

        def prep(idx, b):
            for g in range(CH // 16):
                _unpack_chunk(cbuf, half, idx * CH, srcb2[b], dstb2[b], g)

        def scale_fn(idx, b, r):
            return plsc.load_gather(
                sbuf, [h16, jnp.broadcast_to(idx * CH + r, (16,))])

        _stream_half_db(h_hbm, out_hbm, acc_sh, rows2, srcb2, dstb2, sem2,
                        ssem2, half, nch, cid, sid, prep, scale_fn)


def _gcn_call(src, dst, w, dis, h):
    k = functools.partial(
        pl.kernel,
        out_type=jax.ShapeDtypeStruct((NC, 2, HN, D), F32),
        mesh=_mesh(),
        compiler_params=pltpu.CompilerParams(needs_layout_passes=False),
        scratch_types=[
            pltpu.VMEM((ECH,), I32),
            pltpu.VMEM((ECH,), I32),
            pltpu.VMEM((ECH,), F32),
            pltpu.VMEM((NP,), F32),
            pltpu.VMEM((2, EPTP), I32),
            pltpu.VMEM((2, EPTP), F32),
            pltpu.VMEM((2, 48), I32),
            pltpu.VMEM((2, 48), F32),
            pltpu.VMEM((CH, D), F32),
            pltpu.VMEM((CH, D), F32),
            pltpu.VMEM((CH,), I32),
            pltpu.VMEM((CH,), I32),
            pltpu.VMEM((CH,), I32),
            pltpu.VMEM((CH,), I32),
            pltpu.SemaphoreType.DMA,
            pltpu.SemaphoreType.DMA,
            pltpu.SemaphoreType.DMA,
            pltpu.SemaphoreType.DMA,
            pltpu.VMEM_SHARED((ACC_R, D), F32),
        ],
    )(_gcn_body)
    return k(src, dst, w, dis, h)


def _amax_body(src_hbm, dst_hbm, asrc_hbm, adst_hbm, out_hbm,
               srcv, dstv, asv, adv, loc, kbuf, vbuf, shared, stripebuf, accb):
    cid = lax.axis_index("c")
    sid = lax.axis_index("s")
    wid = cid * NS + sid
    pltpu.sync_copy(src_hbm.at[wid], srcv)
    pltpu.sync_copy(dst_hbm.at[wid], dstv)
    pltpu.sync_copy(asrc_hbm, asv)
    pltpu.sync_copy(adst_hbm, adv)
    _fill_np(loc, -1e30)

    def body(j, _):
        for g in range(CH // 16):
            s16 = srcv[j, pl.ds(g * 16, 16)]
            d16 = dstv[j, pl.ds(g * 16, 16)]
            a = plsc.load_gather(asv, [s16]) + plsc.load_gather(adv, [d16])
            a = jnp.where(a > 0, a, 0.2 * a)
            ks, tot, em = _seg_combine(kbuf, vbuf, d16, a, jnp.maximum)
            cur = plsc.load_gather(loc, [ks])
            plsc.store_scatter(loc, [ks], jnp.maximum(cur, tot), mask=em)
        return 0

    lax.fori_loop(0, NCH, body, 0)
    _combine_tiles(loc, shared, stripebuf, accb, out_hbm, cid, sid, jnp.maximum)


def _amax_call(src_r, dst_r, asrc, adst):
    k = functools.partial(
        pl.kernel,
        out_type=jax.ShapeDtypeStruct((NC, NP), F32),
        mesh=_mesh(),
        compiler_params=pltpu.CompilerParams(needs_layout_passes=False),
        scratch_types=[
            pltpu.VMEM((NCH, CH), I32),
            pltpu.VMEM((NCH, CH), I32),
            pltpu.VMEM((NP,), F32),
            pltpu.VMEM((NP,), F32),
            pltpu.VMEM((NP,), F32),
            pltpu.VMEM((16,), I32),
            pltpu.VMEM((16,), F32),
            pltpu.VMEM_SHARED((NS, NP), F32),
            pltpu.VMEM((NS, STR), F32),
            pltpu.VMEM((STR,), F32),
        ],
    )(_amax_body)
    return k(src_r, dst_r, asrc, adst)


def _gat_body(src_hbm, dst_hbm, asrc_hbm, adst_hbm, amax_hbm, h2_hbm,
              outacc_hbm, outden_hbm,
              srcc, dstc, asv, adv, amv, cbuf, pendc,
              rowsA, rowsB, srcbA, srcbB, dstbA, dstbB, scalebA, scalebB,
              denloc, kbuf, vbuf, semA, semB, ssemA, ssemB, acc_sh):
    cid = lax.axis_index("c")
    sid = lax.axis_index("s")
    wid = cid * NS + sid
    pltpu.sync_copy(asrc_hbm, asv)
    pltpu.sync_copy(adst_hbm, adv)
    pltpu.sync_copy(amax_hbm, amv)
    _fill_np(denloc, 0.0)

    def outer(r, carry):
        base = wid * EPT + r * ECH
        pltpu.sync_copy(src_hbm.at[pl.ds(base, ECH)], srcc)
        pltpu.sync_copy(dst_hbm.at[pl.ds(base, ECH)], dstc)

        def cb(g, carry):
            c0, p0, c1, p1 = carry
            s16 = srcc[pl.ds(g * 16, 16)]
            d16 = dstc[pl.ds(g * 16, 16)]
            c0, p0 = _compact_store(cbuf, pendc, 0, c0, p0, d16, s16)
            c1, p1 = _compact_store(cbuf, pendc, 1, c1, p1, d16, s16)
            return c0, p0, c1, p1

        return lax.fori_loop(0, ECH // 16, cb, carry)

    z = jnp.int32(0)
    c0, p0, c1, p1 = lax.fori_loop(0, NEC, outer, (z, z, z, z))
    rows2, srcb2, dstb2, sem2 = ((rowsA, rowsB), (srcbA, srcbB),
                                 (dstbA, dstbB), (semA, semB))
    ssem2 = (ssemA, ssemB)
    scaleb2 = (scalebA, scalebB)
    for half, nch in ((0, _finalize_compacted(cbuf, pendc, 0, c0, p0)),
                      (1, _finalize_compacted(cbuf, pendc, 1, c1, p1))):

        def prep(idx, b):
            for g in range(CH // 16):
                s16, r16 = _unpack_chunk(cbuf, half, idx * CH,
                                         srcb2[b], dstb2[b], g)
                pad = r16 == DUMMY
                dabs = jnp.where(pad, 0, r16 + half * HN)
                a = (plsc.load_gather(asv, [s16])
                     + plsc.load_gather(adv, [dabs]))
                a = jnp.where(a > 0, a, 0.2 * a)
                e = jnp.exp(a - plsc.load_gather(amv, [dabs]))
                e = jnp.where(pad, 0.0, e)
                scaleb2[b][pl.ds(g * 16, 16)] = e
                ks, tot, em = _seg_combine(kbuf, vbuf, dabs, e,
                                           lambda x, y: x + y)
                plsc.addupdate_scatter(denloc, [ks], tot, mask=em)

        def scale_fn(idx, b, r):
            return plsc.load_gather(scaleb2[b], [jnp.broadcast_to(r, (16,))])

        _stream_half_db(h2_hbm, outacc_hbm, acc_sh, rows2, srcb2, dstb2,
                        sem2, ssem2, half, nch, cid, sid, prep, scale_fn)
    pltpu.sync_copy(denloc, outden_hbm.at[cid, sid])


def _gat_call(src, dst, asrc, adst, amax, h2):
    k = functools.partial(
        pl.kernel,
        out_type=[jax.ShapeDtypeStruct((NC, 2, HN, D), F32),
                  jax.ShapeDtypeStruct((NC, NS, NP), F32)],
        mesh=_mesh(),
        compiler_params=pltpu.CompilerParams(needs_layout_passes=False),
        scratch_types=[
            pltpu.VMEM((ECH,), I32),
            pltpu.VMEM((ECH,), I32),
            pltpu.VMEM((NP,), F32),
            pltpu.VMEM((NP,), F32),
            pltpu.VMEM((NP,), F32),
            pltpu.VMEM((2, EPTP), I32),
            pltpu.VMEM((2, 48), I32),
            pltpu.VMEM((CH, D), F32),
            pltpu.VMEM((CH, D), F32),
            pltpu.VMEM((CH,), I32),
            pltpu.VMEM((CH,), I32),
            pltpu.VMEM((CH,), I32),
            pltpu.VMEM((CH,), I32),
            pltpu.VMEM((CH,), F32),
            pltpu.VMEM((CH,), F32),
            pltpu.VMEM((NP,), F32),
            pltpu.VMEM((16,), I32),
            pltpu.VMEM((16,), F32),
            pltpu.SemaphoreType.DMA,
            pltpu.SemaphoreType.DMA,
            pltpu.SemaphoreType.DMA,
            pltpu.SemaphoreType.DMA,
            pltpu.VMEM_SHARED((ACC_R, D), F32),
        ],
    )(_gat_body)
    return k(src, dst, asrc, adst, amax, h2)


def _mm_body(x_ref, w_ref, o_ref):
    o_ref[...] = jnp.dot(x_ref[...], w_ref[...],
                         preferred_element_type=jnp.float32)


def _matmul(xp, W):
    return pl.pallas_call(
        _mm_body,
        grid=(NB,),
        in_specs=[pl.BlockSpec((BR, D), lambda i: (i, 0)),
                  pl.BlockSpec((D, D), lambda i: (0, 0))],
        out_specs=pl.BlockSpec((BR, D), lambda i: (i, 0)),
        out_shape=jax.ShapeDtypeStruct((NP, D), F32),
    )(xp, W)


def _tc2_body(x_ref, h_ref, a0_ref, a1_ref, dis_ref,
              b1_ref, gnw_ref, gnb_ref, gnms_ref, Wg_ref, as_ref, ad_ref,
              x1_ref, h2_ref, asv_ref, adv_ref, sum_ref, sq_ref):
    p = pl.program_id(0)
    i = pl.program_id(1)
    disv = dis_ref[...][:, None]
    acc = a0_ref[...] + a1_ref[...]
    t = disv * (acc + disv * h_ref[...]) + b1_ref[...][None, :]
    rid = i * BR + lax.broadcasted_iota(I32, (BR, 1), 0)
    t = jnp.where(rid < N, t, 0.0)

    @pl.when(p == 0)
    def _():
        @pl.when(i == 0)
        def _():
            sum_ref[...] = jnp.zeros_like(sum_ref)
            sq_ref[...] = jnp.zeros_like(sq_ref)

        sum_ref[...] += jnp.sum(t, axis=0, keepdims=True)
        sq_ref[...] += jnp.sum(t * t, axis=0, keepdims=True)

    @pl.when(p == 1)
    def _():
        ms = gnms_ref[...][None, :]
        mean = sum_ref[...] / N
        e2 = sq_ref[...] / N
        var = e2 - (2.0 - ms) * ms * mean * mean
        gn = gnw_ref[...][None, :] * (t - ms * mean) / jnp.sqrt(var + 1e-5) \
            + gnb_ref[...][None, :]
        l = jnp.where(gn > 0, gn, 0.01 * gn)
        x1 = x_ref[...] + l
        x1_ref[...] = x1
        h2 = jnp.dot(x1, Wg_ref[...], preferred_element_type=jnp.float32)
        h2_ref[...] = h2
        asv_ref[...] = jnp.sum(h2 * as_ref[...][None, :], axis=1)
        adv_ref[...] = jnp.sum(h2 * ad_ref[...][None, :], axis=1)


def _tc2_call(xp, h, a0, a1, dis, b1, gn_w, gn_b, gn_ms, Wg,
              att_src, att_dst):
    mat = lambda: pl.BlockSpec((BR, D), lambda p, i: (i, 0))
    vec = lambda: pl.BlockSpec((BR,), lambda p, i: (i,))
    dvec = lambda: pl.BlockSpec((D,), lambda p, i: (0,))
    return pl.pallas_call(
        _tc2_body,
        grid=(2, NB),
        in_specs=[mat(), mat(), mat(), mat(), vec(),
                  dvec(), dvec(), dvec(), dvec(),
                  pl.BlockSpec((D, D), lambda p, i: (0, 0)), dvec(), dvec()],
        out_specs=[mat(), mat(), vec(), vec()],
        out_shape=[jax.ShapeDtypeStruct((NP, D), F32),
                   jax.ShapeDtypeStruct((NP, D), F32),
                   jax.ShapeDtypeStruct((NP,), F32),
                   jax.ShapeDtypeStruct((NP,), F32)],
        scratch_shapes=[pltpu.VMEM((1, D), F32), pltpu.VMEM((1, D), F32)],
    )(xp, h, a0, a1, dis, b1, gn_w, gn_b, gn_ms, Wg, att_src, att_dst)


def _tc3_body(x1_ref, h2_ref, g0_ref, g1_ref, es_ref,
              den_ref, bg_ref, gnw_ref, gnb_ref, gnms_ref, out_ref,
              sum_ref, sq_ref):
    p = pl.program_id(0)
    i = pl.program_id(1)
    esv = es_ref[...][:, None]
    denv = (jnp.sum(den_ref[...], axis=0) + es_ref[...])[:, None]
    acc = g0_ref[...] + g1_ref[...]
    t = (acc + h2_ref[...] * esv) / (denv + 1e-16) + bg_ref[...][None, :]
    rid = i * BR + lax.broadcasted_iota(I32, (BR, 1), 0)
    t = jnp.where(rid < N, t, 0.0)

    @pl.when(p == 0)
    def _():
        @pl.when(i == 0)
        def _():
            sum_ref[...] = jnp.zeros_like(sum_ref)
            sq_ref[...] = jnp.zeros_like(sq_ref)

        sum_ref[...] += jnp.sum(t, axis=0, keepdims=True)
        sq_ref[...] += jnp.sum(t * t, axis=0, keepdims=True)

    @pl.when(p == 1)
    def _():
        ms = gnms_ref[...][None, :]
        mean = sum_ref[...] / N
        e2 = sq_ref[...] / N
        var = e2 - (2.0 - ms) * ms * mean * mean
        gn = gnw_ref[...][None, :] * (t - ms * mean) / jnp.sqrt(var + 1e-5) \
            + gnb_ref[...][None, :]
        l = jnp.where(gn > 0, gn, 0.01 * gn)
        out_ref[...] = x1_ref[...] + l


def _tc3_call(x1, h2, g0, g1, es, den, bg, gn_w, gn_b, gn_ms):
    mat = lambda: pl.BlockSpec((BR, D), lambda p, i: (i, 0))
    vec = lambda: pl.BlockSpec((BR,), lambda p, i: (i,))
    dvec = lambda: pl.BlockSpec((D,), lambda p, i: (0,))
    return pl.pallas_call(
        _tc3_body,
        grid=(2, NB),
        in_specs=[mat(), mat(), mat(), mat(), vec(),
                  pl.BlockSpec((NW, BR), lambda p, i: (0, i)),
                  dvec(), dvec(), dvec(), dvec()],
        out_specs=mat(),
        out_shape=jax.ShapeDtypeStruct((NP, D), F32),
        scratch_shapes=[pltpu.VMEM((1, D), F32), pltpu.VMEM((1, D), F32)],
    )(x1, h2, g0, g1, es, den, bg, gn_w, gn_b, gn_ms)


def kernel(x, edges, weight, W1, b1, gn_w, gn_b, gn_ms, Wg, att_src, att_dst,
           bg):
    xp = jnp.zeros((NP, D), F32).at[:N].set(x)
    src_r = edges[0].reshape(NW, NCH, CH)
    dst_r = edges[1].reshape(NW, NCH, CH)
    w_r = weight.reshape(NW, NCH, CH)

    h = _matmul(xp, W1)
    degp = _deg_call(dst_r, w_r)
    deg = degp[0] + degp[1] + 1.0
    dis = jnp.where(deg > 0, 1.0 / jnp.sqrt(deg), 0.0)
    accA = _gcn_call(edges[0], edges[1], weight, dis, h).reshape(NC, NP, D)
    x1, h2, asv, adv = _tc2_call(xp, h, accA[0], accA[1], dis, b1,
                                 gn_w, gn_b, gn_ms, Wg, att_src, att_dst)

    amaxp = _amax_call(src_r, dst_r, asv, adv)
    aself = asv + adv
    aself = jnp.where(aself > 0, aself, 0.2 * aself)
    amax = jnp.maximum(jnp.maximum(amaxp[0], amaxp[1]), aself)
    accG, denp = _gat_call(edges[0], edges[1], asv, adv, amax, h2)
    accG = accG.reshape(NC, NP, D)
    es = jnp.exp(aself - amax)
    out = _tc3_call(x1, h2, accG[0], accG[1], es, denp.reshape(NW, NP), bg,
                    gn_w, gn_b, gn_ms)
    return out[:N]
